# baseline (speedup 1.0000x reference)
import numpy as np
import jax
import jax.numpy as jnp
from jax import lax
from jax.experimental import pallas as pl
from jax.experimental.pallas import tpu as pltpu
from jax.experimental.pallas import tpu_sc as plsc

_B, _C, _F, _T = 64, 1, 128, 3000
_FREQ_MASK_PARAM = 27
_TIME_MASK_PARAM = 100
_NUM_FREQ_MASKS = 2
_NUM_TIME_MASKS = 2

_R = _B * _F
_TP = 3008
_NV = _TP // 16
_GR = 4
_NG = _F // _GR


def _static_masks():
    rng = np.random.RandomState(0)
    partner = np.empty(_B, dtype=np.int64)
    for i in range(_B):
        j = int(rng.randint(0, _B - 1))
        if j >= i:
            j += 1
        partner[i] = j
    fmask = np.zeros((_B, _F), dtype=bool)
    tmask = np.zeros((_B, _T), dtype=bool)
    for i in range(_B):
        for _ in range(_NUM_FREQ_MASKS):
            f = int(rng.randint(0, _FREQ_MASK_PARAM + 1))
            if f == 0:
                continue
            f0 = int(rng.randint(0, _F - f + 1))
            fmask[i, f0:f0 + f] = True
        for _ in range(_NUM_TIME_MASKS):
            t = int(rng.randint(0, _TIME_MASK_PARAM + 1))
            if t == 0:
                continue
            t0 = int(rng.randint(0, _T - t + 1))
            tmask[i, t0:t0 + t] = True
    return partner, fmask, tmask


_PARTNER, _FMASK, _TMASK = _static_masks()


def _build_tables():
    wcol = np.zeros((_B, _TP), np.float32)
    wcol[:, :_T] = 0.5 * _TMASK
    wsp = np.broadcast_to(
        (0.5 * _FMASK.astype(np.float32)).reshape(_R, 1),
        (_R, 16)).reshape(-1).copy()
    return wcol, wsp


def _sc_body(x_hbm, wcol_hbm, wsp_hbm, out_hbm,
             xb, yb, ob, wc, ws,
             sx0, sx1, sy0, sy1, so0, so1):
    sx = (sx0, sx1)
    sy = (sy0, sy1)
    so = (so0, so1)
    wid = lax.axis_index("s") * 2 + lax.axis_index("c")
    gsz = _GR * _TP

    def x_copy(row0, g, par, r):
        return pltpu.make_async_copy(
            x_hbm.at[pl.ds((row0 + _GR * g + r) * _T, _T)],
            xb.at[pl.ds(par * gsz + r * _TP, _T)],
            sx[par])

    def y_copy(prow0, g, par, r):
        return pltpu.make_async_copy(
            x_hbm.at[pl.ds((prow0 + _GR * g + r) * _T, _T)],
            yb.at[pl.ds(par * gsz + r * _TP, _T)],
            sy[par])

    def o_copy(row0, g, par):
        cps = []
        for r in range(_GR):
            cps.append(pltpu.make_async_copy(
                ob.at[pl.ds(par * gsz + r * _TP, _T)],
                out_hbm.at[pl.ds((row0 + _GR * g + r) * _T, _T)],
                so[par]))
        return cps

    def issue_in(row0, prow0, g, par):
        for r in range(_GR):
            x_copy(row0, g, par, r).start()
            y_copy(prow0, g, par, r).start()

    for q in range(2):
        i = wid * 2 + q
        row0 = i * _F
        prow0 = lax.switch(
            i, [lambda p=p: jnp.int32(p * _F) for p in _PARTNER.tolist()])
        pltpu.sync_copy(wcol_hbm.at[pl.ds(i * _TP, _TP)], wc)
        pltpu.sync_copy(wsp_hbm.at[pl.ds(row0 * 16, _F * 16)], ws)

        issue_in(row0, prow0, 0, 0)
        issue_in(row0, prow0, 1, 1)

        def step(g2, par, row0=row0, prow0=prow0):
            g = 2 * g2 + par
            for r in range(_GR):
                x_copy(row0, g, par, r).wait()
                y_copy(prow0, g, par, r).wait()

            @pl.when(g2 >= 1)
            def _():
                for cp in o_copy(row0, g - 2, par):
                    cp.wait()

            ws_rows = [ws[pl.ds((_GR * g + r) * 16, 16)] for r in range(_GR)]

            def vbody(v, carry):
                off = v * 16
                wcv = wc[pl.ds(off, 16)]
                for r in range(_GR):
                    base = par * gsz + r * _TP + off
                    xv = xb[pl.ds(base, 16)]
                    yv = yb[pl.ds(base, 16)]
                    w = jnp.maximum(wcv, ws_rows[r])
                    ob[pl.ds(base, 16)] = xv + w * (yv - xv)
                return carry

            lax.fori_loop(0, _NV, vbody, 0, unroll=4)
            for cp in o_copy(row0, g, par):
                cp.start()

            @pl.when(g2 < _NG // 2 - 1)
            def _():
                issue_in(row0, prow0, g + 2, par)

        def g2body(g2, carry):
            step(g2, 0)
            step(g2, 1)
            return carry

        lax.fori_loop(0, _NG // 2, g2body, 0)
        for cp in o_copy(row0, _NG - 2, 0):
            cp.wait()
        for cp in o_copy(row0, _NG - 1, 1):
            cp.wait()


def kernel(x):
    wcol_np, wsp_np = _build_tables()
    x2 = x.reshape(_R * _T)
    mesh = plsc.VectorSubcoreMesh(core_axis_name="c", subcore_axis_name="s")
    aug2 = pl.kernel(
        _sc_body,
        mesh=mesh,
        out_type=jax.ShapeDtypeStruct((_R * _T,), jnp.float32),
        scratch_types=[
            pltpu.VMEM((2 * _GR * _TP,), jnp.float32),
            pltpu.VMEM((2 * _GR * _TP,), jnp.float32),
            pltpu.VMEM((2 * _GR * _TP,), jnp.float32),
            pltpu.VMEM((_TP,), jnp.float32),
            pltpu.VMEM((_F * 16,), jnp.float32),
            pltpu.SemaphoreType.DMA,
            pltpu.SemaphoreType.DMA,
            pltpu.SemaphoreType.DMA,
            pltpu.SemaphoreType.DMA,
            pltpu.SemaphoreType.DMA,
            pltpu.SemaphoreType.DMA,
        ],
    )(x2, jnp.asarray(wcol_np.reshape(-1)), jnp.asarray(wsp_np))
    aug = aug2.reshape(_B, _C, _F, _T)
    fm = jnp.asarray(_FMASK)
    tm = jnp.asarray(_TMASK)
    partner_idx = jnp.asarray(_PARTNER, dtype=jnp.int64)
    return (aug, fm, tm, partner_idx)


# speedup vs baseline: 1.2170x; 1.2170x over previous
import numpy as np
import jax
import jax.numpy as jnp
from jax import lax
from jax.experimental import pallas as pl
from jax.experimental.pallas import tpu as pltpu
from jax.experimental.pallas import tpu_sc as plsc

_B, _C, _F, _T = 64, 1, 128, 3000
_FREQ_MASK_PARAM = 27
_TIME_MASK_PARAM = 100
_NUM_FREQ_MASKS = 2
_NUM_TIME_MASKS = 2

_TR = (_B * _F) // 8
_KA = 12
_KB = 11
_TAIL0 = 2944
_TAILW = _T - _TAIL0
_TP = 3008


def _static_masks():
    rng = np.random.RandomState(0)
    partner = np.empty(_B, dtype=np.int64)
    for i in range(_B):
        j = int(rng.randint(0, _B - 1))
        if j >= i:
            j += 1
        partner[i] = j
    fmask = np.zeros((_B, _F), dtype=bool)
    tmask = np.zeros((_B, _T), dtype=bool)
    for i in range(_B):
        for _ in range(_NUM_FREQ_MASKS):
            f = int(rng.randint(0, _FREQ_MASK_PARAM + 1))
            if f == 0:
                continue
            f0 = int(rng.randint(0, _F - f + 1))
            fmask[i, f0:f0 + f] = True
        for _ in range(_NUM_TIME_MASKS):
            t = int(rng.randint(0, _TIME_MASK_PARAM + 1))
            if t == 0:
                continue
            t0 = int(rng.randint(0, _T - t + 1))
            tmask[i, t0:t0 + t] = True
    return partner, fmask, tmask


_PARTNER, _FMASK, _TMASK = _static_masks()


def _build_tables():
    wcol = np.zeros((_B * _TP,), np.float32)
    for i in range(_B):
        wcol[i * _TP:i * _TP + _T] = 0.5 * _TMASK[i]
    wsp = np.broadcast_to(
        (0.5 * _FMASK.astype(np.float32)).reshape(_B * _F, 1),
        (_B * _F, 16)).reshape(-1).copy()
    return wcol, wsp


def _sc_body(x_hbm, wcol_hbm, wsp_hbm, out_hbm,
             xa, ya, oa, xb, yb, ob, xt, yt, ot, wc, ws,
             sax, say, sao, sbx, sby, sbo):
    wid = lax.axis_index("s") * 2 + lax.axis_index("c")

    def a_in(tk, ptk):
        for t in range(_KA):
            pltpu.make_async_copy(
                x_hbm.at[tk, :, pl.ds(t * 128, 128)],
                xa.at[pl.ds(t * 8, 8), :], sax).start()
            pltpu.make_async_copy(
                x_hbm.at[ptk, :, pl.ds(t * 128, 128)],
                ya.at[pl.ds(t * 8, 8), :], say).start()

    def a_in_wait(tk, ptk):
        for t in range(_KA):
            pltpu.make_async_copy(
                x_hbm.at[tk, :, pl.ds(t * 128, 128)],
                xa.at[pl.ds(t * 8, 8), :], sax).wait()
            pltpu.make_async_copy(
                x_hbm.at[ptk, :, pl.ds(t * 128, 128)],
                ya.at[pl.ds(t * 8, 8), :], say).wait()

    def a_out(tk, start):
        for t in range(_KA):
            cp = pltpu.make_async_copy(
                oa.at[pl.ds(t * 8, 8), :],
                out_hbm.at[tk, :, pl.ds(t * 128, 128)], sao)
            cp.start() if start else cp.wait()

    def b_in(tk, ptk, start):
        for t in range(_KB):
            for src, dst, sem in ((x_hbm, xb, sbx), (x_hbm, yb, sby)):
                base = tk if dst is xb else ptk
                cp = pltpu.make_async_copy(
                    src.at[base, :, pl.ds(1536 + t * 128, 128)],
                    dst.at[pl.ds(t * 8, 8), :], sem)
                cp.start() if start else cp.wait()
        for r in range(8):
            for src, dst, sem in ((x_hbm, xt, sbx), (x_hbm, yt, sby)):
                base = tk if dst is xt else ptk
                cp = pltpu.make_async_copy(
                    src.at[base, r, pl.ds(_TAIL0, _TAILW)],
                    dst.at[r, pl.ds(0, _TAILW)], sem)
                cp.start() if start else cp.wait()

    def b_out(tk, start):
        for t in range(_KB):
            cp = pltpu.make_async_copy(
                ob.at[pl.ds(t * 8, 8), :],
                out_hbm.at[tk, :, pl.ds(1536 + t * 128, 128)], sbo)
            cp.start() if start else cp.wait()
        for r in range(8):
            cp = pltpu.make_async_copy(
                ot.at[r, pl.ds(0, _TAILW)],
                out_hbm.at[tk, r, pl.ds(_TAIL0, _TAILW)], sbo)
            cp.start() if start else cp.wait()

    def blend_chunk(k8, nt, colbase, src_x, src_y, dst):
        def tbody(t, carry):
            wsl = [ws[pl.ds((k8 * 8 + r) * 16, 16)] for r in range(8)]
            for r in range(8):
                for cv in range(8):
                    co = colbase + cv * 16
                    wcv = wc[pl.ds(t * 128 + co, 16)]
                    w = jnp.maximum(wcv, wsl[r])
                    xv = src_x[t * 8 + r, pl.ds(cv * 16, 16)]
                    yv = src_y[t * 8 + r, pl.ds(cv * 16, 16)]
                    dst[t * 8 + r, pl.ds(cv * 16, 16)] = xv + w * (yv - xv)
            return carry
        lax.fori_loop(0, nt, tbody, 0)

    def blend_tail(k8):
        for r in range(8):
            wsr = ws[pl.ds((k8 * 8 + r) * 16, 16)]
            for cv in range(4):
                wcv = wc[pl.ds(_TAIL0 + cv * 16, 16)]
                w = jnp.maximum(wcv, wsr)
                xv = xt[r, pl.ds(cv * 16, 16)]
                yv = yt[r, pl.ds(cv * 16, 16)]
                ot[r, pl.ds(cv * 16, 16)] = xv + w * (yv - xv)

    for q in range(2):
        i = wid * 2 + q
        tk0 = i * 16
        ptk0 = lax.switch(
            i, [lambda p=p: jnp.int32(p * 16) for p in _PARTNER.tolist()])
        pltpu.sync_copy(wcol_hbm.at[pl.ds(i * _TP, _TP)], wc)
        pltpu.sync_copy(wsp_hbm.at[pl.ds(i * _F * 16, _F * 16)], ws)

        a_in(tk0, ptk0)
        b_in(tk0, ptk0, True)

        def k8body(k8, carry, tk0=tk0, ptk0=ptk0):
            tk = tk0 + k8
            ptk = ptk0 + k8
            a_in_wait(tk, ptk)

            @pl.when(k8 >= 1)
            def _():
                a_out(tk - 1, False)

            blend_chunk(k8, _KA, 0, xa, ya, oa)
            a_out(tk, True)

            @pl.when(k8 < 15)
            def _():
                a_in(tk + 1, ptk + 1)

            b_in(tk, ptk, False)

            @pl.when(k8 >= 1)
            def _():
                b_out(tk - 1, False)

            blend_chunk(k8, _KB, 1536, xb, yb, ob)
            blend_tail(k8)
            b_out(tk, True)

            @pl.when(k8 < 15)
            def _():
                b_in(tk + 1, ptk + 1, True)

            return carry

        lax.fori_loop(0, 16, k8body, 0)
        a_out(tk0 + 15, False)
        b_out(tk0 + 15, False)


def kernel(x):
    wcol_np, wsp_np = _build_tables()
    x4 = x.reshape(_TR, 8, _T)
    mesh = plsc.VectorSubcoreMesh(core_axis_name="c", subcore_axis_name="s")
    aug4 = pl.kernel(
        _sc_body,
        mesh=mesh,
        out_type=jax.ShapeDtypeStruct((_TR, 8, _T), jnp.float32),
        scratch_types=[
            pltpu.VMEM((_KA * 8, 128), jnp.float32),
            pltpu.VMEM((_KA * 8, 128), jnp.float32),
            pltpu.VMEM((_KA * 8, 128), jnp.float32),
            pltpu.VMEM((_KB * 8, 128), jnp.float32),
            pltpu.VMEM((_KB * 8, 128), jnp.float32),
            pltpu.VMEM((_KB * 8, 128), jnp.float32),
            pltpu.VMEM((8, 128), jnp.float32),
            pltpu.VMEM((8, 128), jnp.float32),
            pltpu.VMEM((8, 128), jnp.float32),
            pltpu.VMEM((_TP,), jnp.float32),
            pltpu.VMEM((_F * 16,), jnp.float32),
            pltpu.SemaphoreType.DMA,
            pltpu.SemaphoreType.DMA,
            pltpu.SemaphoreType.DMA,
            pltpu.SemaphoreType.DMA,
            pltpu.SemaphoreType.DMA,
            pltpu.SemaphoreType.DMA,
        ],
    )(x4, jnp.asarray(wcol_np), jnp.asarray(wsp_np))
    aug = aug4.reshape(_B, _C, _F, _T)
    fm = jnp.asarray(_FMASK)
    tm = jnp.asarray(_TMASK)
    partner_idx = jnp.asarray(_PARTNER, dtype=jnp.int64)
    return (aug, fm, tm, partner_idx)


# speedup vs baseline: 2.2920x; 1.8833x over previous
import numpy as np
import jax
import jax.numpy as jnp
from jax.experimental import pallas as pl
from jax.experimental.pallas import tpu as pltpu

_B, _C, _F, _T = 64, 1, 128, 3000
_FREQ_MASK_PARAM = 27
_TIME_MASK_PARAM = 100
_NUM_FREQ_MASKS = 2
_NUM_TIME_MASKS = 2


def _static_masks():
    rng = np.random.RandomState(0)
    partner = np.empty(_B, dtype=np.int64)
    for i in range(_B):
        j = int(rng.randint(0, _B - 1))
        if j >= i:
            j += 1
        partner[i] = j
    fmask = np.zeros((_B, _F), dtype=bool)
    tmask = np.zeros((_B, _T), dtype=bool)
    for i in range(_B):
        for _ in range(_NUM_FREQ_MASKS):
            f = int(rng.randint(0, _FREQ_MASK_PARAM + 1))
            if f == 0:
                continue
            f0 = int(rng.randint(0, _F - f + 1))
            fmask[i, f0:f0 + f] = True
        for _ in range(_NUM_TIME_MASKS):
            t = int(rng.randint(0, _TIME_MASK_PARAM + 1))
            if t == 0:
                continue
            t0 = int(rng.randint(0, _T - t + 1))
            tmask[i, t0:t0 + t] = True
    return partner, fmask, tmask


_PARTNER, _FMASK, _TMASK = _static_masks()


def _mix_body(p_ref, fm_ref, tm_ref, xe_ref, xo_ref, ye_ref, yo_ref, o_ref):
    for half, (x_ref, y_ref) in enumerate(((xe_ref, ye_ref), (xo_ref, yo_ref))):
        fm = fm_ref[half, 0, :]
        tm = tm_ref[half, 0, :]
        w = jnp.maximum(fm[:, None], tm[None, :])
        xv = x_ref[0, 0]
        yv = y_ref[0, 0]
        o_ref[half, 0] = xv + (0.5 * w) * (yv - xv)


def kernel(x):
    fm_f = jnp.asarray(_FMASK.astype(np.float32)).reshape(_B, 1, _F)
    tm_f = jnp.asarray(_TMASK.astype(np.float32)).reshape(_B, 1, _T)
    aug = pl.pallas_call(
        _mix_body,
        grid_spec=pltpu.PrefetchScalarGridSpec(
            num_scalar_prefetch=1,
            grid=(_B // 2,),
            in_specs=[
                pl.BlockSpec((2, 1, _F), lambda i, p: (i, 0, 0)),
                pl.BlockSpec((2, 1, _T), lambda i, p: (i, 0, 0)),
                pl.BlockSpec((1, 1, _F, _T), lambda i, p: (2 * i, 0, 0, 0)),
                pl.BlockSpec((1, 1, _F, _T), lambda i, p: (2 * i + 1, 0, 0, 0)),
                pl.BlockSpec((1, 1, _F, _T), lambda i, p: (p[2 * i], 0, 0, 0)),
                pl.BlockSpec((1, 1, _F, _T), lambda i, p: (p[2 * i + 1], 0, 0, 0)),
            ],
            out_specs=pl.BlockSpec((2, 1, _F, _T), lambda i, p: (i, 0, 0, 0)),
        ),
        out_shape=jax.ShapeDtypeStruct((_B, _C, _F, _T), x.dtype),
    )(jnp.asarray(_PARTNER.astype(np.int32)), fm_f, tm_f, x, x, x, x)
    fm = jnp.asarray(_FMASK)
    tm = jnp.asarray(_TMASK)
    partner_idx = jnp.asarray(_PARTNER, dtype=jnp.int64)
    return (aug, fm, tm, partner_idx)


# speedup vs baseline: 2.3334x; 1.0181x over previous
import numpy as np
import jax
import jax.numpy as jnp
from jax.experimental import pallas as pl
from jax.experimental.pallas import tpu as pltpu

_B, _C, _F, _T = 64, 1, 128, 3000
_FREQ_MASK_PARAM = 27
_TIME_MASK_PARAM = 100
_NUM_FREQ_MASKS = 2
_NUM_TIME_MASKS = 2


def _static_masks():
    rng = np.random.RandomState(0)
    partner = np.empty(_B, dtype=np.int64)
    for i in range(_B):
        j = int(rng.randint(0, _B - 1))
        if j >= i:
            j += 1
        partner[i] = j
    fmask = np.zeros((_B, _F), dtype=bool)
    tmask = np.zeros((_B, _T), dtype=bool)
    for i in range(_B):
        for _ in range(_NUM_FREQ_MASKS):
            f = int(rng.randint(0, _FREQ_MASK_PARAM + 1))
            if f == 0:
                continue
            f0 = int(rng.randint(0, _F - f + 1))
            fmask[i, f0:f0 + f] = True
        for _ in range(_NUM_TIME_MASKS):
            t = int(rng.randint(0, _TIME_MASK_PARAM + 1))
            if t == 0:
                continue
            t0 = int(rng.randint(0, _T - t + 1))
            tmask[i, t0:t0 + t] = True
    return partner, fmask, tmask


_PARTNER, _FMASK, _TMASK = _static_masks()


def _mix_body(p_ref, fm_ref, tm_ref, x0_ref, x1_ref, x2_ref, x3_ref,
              y0_ref, y1_ref, y2_ref, y3_ref, o_ref):
    xs = (x0_ref, x1_ref, x2_ref, x3_ref)
    ys = (y0_ref, y1_ref, y2_ref, y3_ref)
    for part in range(4):
        fm = fm_ref[part, 0, :]
        tm = tm_ref[part, 0, :]
        w = jnp.maximum(fm[:, None], tm[None, :])
        xv = xs[part][0, 0]
        yv = ys[part][0, 0]
        o_ref[part, 0] = xv + (0.5 * w) * (yv - xv)


def kernel(x):
    fm_f = jnp.asarray(_FMASK.astype(np.float32)).reshape(_B, 1, _F)
    tm_f = jnp.asarray(_TMASK.astype(np.float32)).reshape(_B, 1, _T)
    aug = pl.pallas_call(
        _mix_body,
        grid_spec=pltpu.PrefetchScalarGridSpec(
            num_scalar_prefetch=1,
            grid=(_B // 4,),
            in_specs=[
                pl.BlockSpec((4, 1, _F), lambda i, p: (i, 0, 0)),
                pl.BlockSpec((4, 1, _T), lambda i, p: (i, 0, 0)),
                pl.BlockSpec((1, 1, _F, _T), lambda i, p: (4 * i, 0, 0, 0)),
                pl.BlockSpec((1, 1, _F, _T), lambda i, p: (4 * i + 1, 0, 0, 0)),
                pl.BlockSpec((1, 1, _F, _T), lambda i, p: (4 * i + 2, 0, 0, 0)),
                pl.BlockSpec((1, 1, _F, _T), lambda i, p: (4 * i + 3, 0, 0, 0)),
                pl.BlockSpec((1, 1, _F, _T), lambda i, p: (p[4 * i], 0, 0, 0)),
                pl.BlockSpec((1, 1, _F, _T), lambda i, p: (p[4 * i + 1], 0, 0, 0)),
                pl.BlockSpec((1, 1, _F, _T), lambda i, p: (p[4 * i + 2], 0, 0, 0)),
                pl.BlockSpec((1, 1, _F, _T), lambda i, p: (p[4 * i + 3], 0, 0, 0)),
            ],
            out_specs=pl.BlockSpec((4, 1, _F, _T), lambda i, p: (i, 0, 0, 0)),
        ),
        out_shape=jax.ShapeDtypeStruct((_B, _C, _F, _T), x.dtype),
    )(jnp.asarray(_PARTNER.astype(np.int32)), fm_f, tm_f, x, x, x, x, x, x, x, x)
    fm = jnp.asarray(_FMASK)
    tm = jnp.asarray(_TMASK)
    partner_idx = jnp.asarray(_PARTNER, dtype=jnp.int64)
    return (aug, fm, tm, partner_idx)
